# baseline (speedup 1.0000x reference)
import functools

import jax
import jax.numpy as jnp
from jax import lax
from jax.experimental import pallas as pl
from jax.experimental.pallas import tpu as pltpu
from jax.experimental.pallas import tpu_sc as plsc

N = 10000
D = 128
NB = D // 16
NW = 32
NPW = 320
NPAD = NW * NPW
CH = 128
PTR_PAD = 24
PTR_LEN = NW * NPW + PTR_PAD
EP = 160000 + N + 7 * N
MAXCH = EP // CH + 1
JW = ((MAXCH + 16 + 7) // 8) * 8
NEG = jnp.float32(-3.0e38)


@functools.cache
def _make_segmax():
    mesh = plsc.VectorSubcoreMesh(core_axis_name="c", subcore_axis_name="s")
    return functools.partial(
        pl.kernel,
        mesh=mesh,
        out_type=jax.ShapeDtypeStruct((NPAD, D), jnp.float32),
        scratch_types=[
            pltpu.VMEM((NPW + PTR_PAD,), jnp.int32),
            pltpu.VMEM((JW,), jnp.int32),
            pltpu.VMEM((CH,), jnp.int32),
            pltpu.VMEM((CH, D), jnp.float32),
            pltpu.VMEM((NPW, D), jnp.float32),
            pltpu.SemaphoreType.DMA,
        ],
    )(_segmax_body)


def _pload(ref, i):
    return ref[pl.ds(i, 16)][0]


def _segmax_body(g_hbm, srcp_hbm, ptr_hbm, jend_hbm, out_hbm,
                 ptr_v, jend_v, idx_v, rows_v, out_v, sem):
    wid = lax.axis_index("s") * 2 + lax.axis_index("c")
    n0 = pl.multiple_of(wid * NPW, 8)
    cnt = jnp.minimum(N - n0, NPW)
    pltpu.sync_copy(ptr_hbm.at[pl.ds(n0, NPW + PTR_PAD)], ptr_v)
    pltpu.sync_copy(jend_hbm.at[wid], jend_v)
    p_start = _pload(ptr_v, 0)
    p_end = _pload(ptr_v, cnt)
    nch = (p_end - p_start + (CH - 1)) // CH
    neg = jnp.full((16,), NEG, dtype=jnp.float32)

    def edge_max(lo, e0, e1, acc):
        def edge_body(e, a):
            k = e - lo
            return tuple(jnp.maximum(a[f], rows_v[k, pl.ds(f * 16, 16)])
                         for f in range(NB))

        return lax.fori_loop(e0, jnp.maximum(e0, e1), edge_body, acc)

    def chunk_body(c, carry):
        jc = carry[0]
        acc = carry[1:]
        lo = pl.multiple_of(p_start + c * CH, 8)
        hi = jnp.minimum(lo + CH, p_end)
        pltpu.sync_copy(srcp_hbm.at[pl.ds(lo, CH)], idx_v)
        pltpu.async_copy(g_hbm.at[idx_v], rows_v, sem).wait()
        je = _pload(jend_v, c)

        def node_body(j, a):
            pv = ptr_v[pl.ds(j, 16)]
            p0, p1 = pv[0], pv[1]
            a = edge_max(lo, jnp.maximum(p0, lo), p1, a)
            for f in range(NB):
                out_v[j, pl.ds(f * 16, 16)] = a[f]
            return tuple(neg for _ in range(NB))

        acc = lax.fori_loop(jc, je, node_body, acc)
        p0 = _pload(ptr_v, je)
        acc = edge_max(lo, jnp.maximum(p0, lo), hi, acc)
        return (je,) + acc

    init = (jnp.int32(0),) + tuple(neg for _ in range(NB))
    lax.fori_loop(0, nch, chunk_body, init)
    pltpu.sync_copy(out_v, out_hbm.at[pl.ds(n0, NPW)])


def _tc_first_body(x_ref, w_ref, dinv_ref, o_ref):
    o_ref[...] = jnp.dot(x_ref[...], w_ref[...],
                         preferred_element_type=jnp.float32) * dinv_ref[...]


def _tc_first(xx, W, dinv2):
    return pl.pallas_call(
        _tc_first_body,
        out_shape=jax.ShapeDtypeStruct((N, W.shape[1]), jnp.float32),
    )(xx, W, dinv2)


def _tc_mid_body(m_ref, dinv_ref, b_ref, w_ref, o_ref):
    h = m_ref[...] * dinv_ref[...] + b_ref[...]
    mu = jnp.mean(h, axis=0, keepdims=True)
    v = jnp.mean((h - mu) ** 2, axis=0, keepdims=True)
    hn = (h - mu) / jnp.sqrt(v + 1e-5)
    a = jnp.where(hn >= 0, hn, 0.02 * hn)
    o_ref[...] = jnp.dot(a, w_ref[...],
                         preferred_element_type=jnp.float32) * dinv_ref[...]


def _tc_mid(m, dinv2, b, W):
    return pl.pallas_call(
        _tc_mid_body,
        out_shape=jax.ShapeDtypeStruct((N, W.shape[1]), jnp.float32),
    )(m, dinv2, b, W)


def _tc_globend_body(m_ref, dinv_ref, b_ref, o_ref):
    h = m_ref[...] * dinv_ref[...] + b_ref[...]
    o_ref[...] = jnp.mean(h, axis=0, keepdims=True)


def _tc_globend(m, dinv2, b):
    return pl.pallas_call(
        _tc_globend_body,
        out_shape=jax.ShapeDtypeStruct((1, D), jnp.float32),
    )(m, dinv2, b)


def _tc_tail0_body(mh_ref, ms_ref, grow_ref, dinv_ref, bh_ref, bs_ref,
                   wa_ref, wb_ref, wc_ref, o_ref):
    hh = mh_ref[...] * dinv_ref[...] + bh_ref[...]
    hs = ms_ref[...] * dinv_ref[...] + bs_ref[...]
    g = (jnp.dot(hh, wa_ref[...], preferred_element_type=jnp.float32)
         + jnp.dot(hs, wb_ref[...], preferred_element_type=jnp.float32)
         + jnp.dot(grow_ref[...], wc_ref[...],
                   preferred_element_type=jnp.float32))
    o_ref[...] = g * dinv_ref[...]


def _tc_tail0(mh, ms, grow, dinv2, bh, bs, wa, wb, wc):
    return pl.pallas_call(
        _tc_tail0_body,
        out_shape=jax.ShapeDtypeStruct((N, D), jnp.float32),
    )(mh, ms, grow, dinv2, bh, bs, wa, wb, wc)


def _tc_final_body(m_ref, dinv_ref, b_ref, o_ref):
    o_ref[...] = jnp.tanh(m_ref[...] * dinv_ref[...] + b_ref[...]) * 0.5


def _tc_final(m, dinv2, b):
    return pl.pallas_call(
        _tc_final_body,
        out_shape=jax.ShapeDtypeStruct((N, D), jnp.float32),
    )(m, dinv2, b)


def _build_csr(edge_index):
    src = edge_index[0].astype(jnp.int32)
    dst = edge_index[1].astype(jnp.int32)
    loop = jnp.arange(N, dtype=jnp.int32)
    src_f = jnp.concatenate([src, loop])
    dst_f = jnp.concatenate([dst, loop])
    ef = src_f.shape[0]
    order = jnp.argsort(dst_f)
    src_s = src_f[order]
    ds = dst_f[order]
    ptr = jnp.searchsorted(ds, jnp.arange(N + 1, dtype=jnp.int32),
                           side='left').astype(jnp.int32)
    deg = ptr[1:] - ptr[:-1]
    dinv = 1.0 / jnp.sqrt(deg.astype(jnp.float32))
    deg_pad = ((deg + 7) // 8) * 8
    ptr_pad = jnp.concatenate([
        jnp.zeros((1,), jnp.int32),
        jnp.cumsum(deg_pad, dtype=jnp.int32)])
    ep = ef + 7 * N
    owner = jnp.repeat(loop, deg_pad, total_repeat_length=ep)
    slots = jnp.arange(ep, dtype=jnp.int32)
    r = slots - ptr_pad[owner]
    isreal = r < deg[owner]
    src_idx = jnp.clip(ptr[owner] + r, 0, ef - 1)
    srcp = jnp.where(isreal, src_s[src_idx], owner)
    srcp = jnp.concatenate([srcp, jnp.zeros((CH,), jnp.int32)])
    ptr_full = jnp.concatenate([
        ptr_pad,
        jnp.broadcast_to(ptr_pad[N:N + 1], (PTR_LEN - (N + 1),))])
    n0_w = jnp.arange(NW, dtype=jnp.int32) * NPW
    cnt_w = jnp.clip(N - n0_w, 0, NPW)
    ps_w = ptr_pad[n0_w]
    pe_w = ptr_pad[n0_w + cnt_w]
    hi = jnp.minimum(ps_w[:, None]
                     + (jnp.arange(JW, dtype=jnp.int32)[None, :] + 1) * CH,
                     pe_w[:, None])
    jend = jnp.searchsorted(ptr_pad, hi, side='right').astype(jnp.int32) - 1
    jend = jnp.clip(jend - n0_w[:, None], 0, cnt_w[:, None])
    return srcp, ptr_full, jend, dinv[:, None]


def kernel(x, edge_index, head_W0, head_b0, head_W1, head_b1, head_W2, head_b2,
           head_W3, head_b3, head_W4, head_b4, skip_W0, skip_b0,
           glob_W0, glob_b0, glob_W1, glob_b1,
           tail_W0, tail_b0, tail_W1, tail_b1):
    srcp, ptr_full, jend, dinv2 = _build_csr(edge_index)

    seg = _make_segmax()

    def agg(g):
        return seg(g, srcp, ptr_full, jend)[:N]

    m = agg(_tc_first(x, head_W0, dinv2))
    for b_prev, W in [(head_b0, head_W1), (head_b1, head_W2),
                      (head_b2, head_W3), (head_b3, head_W4)]:
        m = agg(_tc_mid(m, dinv2, b_prev[None, :], W))
    m_head = m

    m_skip = agg(_tc_first(x, skip_W0, dinv2))

    mg = agg(_tc_first(x, glob_W0, dinv2))
    mg = agg(_tc_mid(mg, dinv2, glob_b0[None, :], glob_W1))
    grow = _tc_globend(mg, dinv2, glob_b1[None, :])

    wa, wb, wc = tail_W0[0:D], tail_W0[D:2 * D], tail_W0[2 * D:3 * D]
    gt = _tc_tail0(m_head, m_skip, grow, dinv2, head_b4[None, :],
                   skip_b0[None, :], wa, wb, wc)
    mt = agg(gt)
    mt = agg(_tc_mid(mt, dinv2, tail_b0[None, :], tail_W1))
    return _tc_final(mt, dinv2, tail_b1[None, :])


# speedup vs baseline: 3.8471x; 3.8471x over previous
import functools

import jax
import jax.numpy as jnp
from jax import lax
from jax.experimental import pallas as pl
from jax.experimental.pallas import tpu as pltpu
from jax.experimental.pallas import tpu_sc as plsc

N = 10000
D = 128
NB = D // 16
NW = 32
NPW = 320
NPAD = NW * NPW
CH = 128
PTR_PAD = 24
PTR_LEN = NW * NPW + PTR_PAD
EP = 160000 + N + 7 * N
MAXCH = EP // CH + 1
JW = ((MAXCH + 16 + 7) // 8) * 8
NEG = jnp.float32(-3.0e38)


@functools.cache
def _make_segmax():
    mesh = plsc.VectorSubcoreMesh(core_axis_name="c", subcore_axis_name="s")
    return functools.partial(
        pl.kernel,
        mesh=mesh,
        out_type=jax.ShapeDtypeStruct((NPAD, D), jnp.float32),
        scratch_types=[
            pltpu.VMEM((NPW + PTR_PAD,), jnp.int32),
            pltpu.VMEM((JW,), jnp.int32),
            pltpu.VMEM((CH,), jnp.int32),
            pltpu.VMEM((CH, D), jnp.float32),
            pltpu.VMEM((NPW, D), jnp.float32),
            pltpu.SemaphoreType.DMA,
        ],
    )(_segmax_body)


def _pload(ref, i):
    return ref[pl.ds(i, 16)][0]


def _segmax_body(g_hbm, srcp_hbm, ptr_hbm, jend_hbm, out_hbm,
                 ptr_v, jend_v, idx_v, rows_v, out_v, sem):
    wid = lax.axis_index("s") * 2 + lax.axis_index("c")
    n0 = pl.multiple_of(wid * NPW, 8)
    cnt = jnp.minimum(N - n0, NPW)
    pltpu.sync_copy(ptr_hbm.at[pl.ds(n0, NPW + PTR_PAD)], ptr_v)
    pltpu.sync_copy(jend_hbm.at[wid], jend_v)
    p_start = _pload(ptr_v, 0)
    p_end = _pload(ptr_v, cnt)
    nch = (p_end - p_start + (CH - 1)) // CH
    neg = jnp.full((16,), NEG, dtype=jnp.float32)

    def edge_max(lo, e0, e1, acc):
        def edge_body(e, a):
            k = e - lo
            return tuple(jnp.maximum(a[f], rows_v[k, pl.ds(f * 16, 16)])
                         for f in range(NB))

        return lax.fori_loop(e0, jnp.maximum(e0, e1), edge_body, acc)

    def chunk_body(c, carry):
        jc = carry[0]
        acc = carry[1:]
        lo = pl.multiple_of(p_start + c * CH, 8)
        hi = jnp.minimum(lo + CH, p_end)
        pltpu.sync_copy(srcp_hbm.at[pl.ds(lo, CH)], idx_v)
        pltpu.async_copy(g_hbm.at[idx_v], rows_v, sem).wait()
        je = _pload(jend_v, c)

        def node_body(j, a):
            pv = ptr_v[pl.ds(j, 16)]
            p0, p1 = pv[0], pv[1]
            a = edge_max(lo, jnp.maximum(p0, lo), p1, a)
            for f in range(NB):
                out_v[j, pl.ds(f * 16, 16)] = a[f]
            return tuple(neg for _ in range(NB))

        acc = lax.fori_loop(jc, je, node_body, acc)
        p0 = _pload(ptr_v, je)
        acc = edge_max(lo, jnp.maximum(p0, lo), hi, acc)
        return (je,) + acc

    init = (jnp.int32(0),) + tuple(neg for _ in range(NB))
    lax.fori_loop(0, nch, chunk_body, init)
    pltpu.sync_copy(out_v, out_hbm.at[pl.ds(n0, NPW)])


def _tc_first_body(x_ref, w_ref, dinv_ref, o_ref):
    o_ref[...] = jnp.dot(x_ref[...], w_ref[...],
                         preferred_element_type=jnp.float32) * dinv_ref[...]


def _tc_first(xx, W, dinv2):
    return pl.pallas_call(
        _tc_first_body,
        out_shape=jax.ShapeDtypeStruct((N, W.shape[1]), jnp.float32),
    )(xx, W, dinv2)


def _tc_mid_body(m_ref, dinv_ref, b_ref, w_ref, o_ref):
    h = m_ref[...] * dinv_ref[...] + b_ref[...]
    mu = jnp.mean(h, axis=0, keepdims=True)
    v = jnp.mean((h - mu) ** 2, axis=0, keepdims=True)
    hn = (h - mu) / jnp.sqrt(v + 1e-5)
    a = jnp.where(hn >= 0, hn, 0.02 * hn)
    o_ref[...] = jnp.dot(a, w_ref[...],
                         preferred_element_type=jnp.float32) * dinv_ref[...]


def _tc_mid(m, dinv2, b, W):
    return pl.pallas_call(
        _tc_mid_body,
        out_shape=jax.ShapeDtypeStruct((N, W.shape[1]), jnp.float32),
    )(m, dinv2, b, W)


def _tc_globend_body(m_ref, dinv_ref, b_ref, o_ref):
    h = m_ref[...] * dinv_ref[...] + b_ref[...]
    o_ref[...] = jnp.mean(h, axis=0, keepdims=True)


def _tc_globend(m, dinv2, b):
    return pl.pallas_call(
        _tc_globend_body,
        out_shape=jax.ShapeDtypeStruct((1, D), jnp.float32),
    )(m, dinv2, b)


def _tc_tail0_body(mh_ref, ms_ref, grow_ref, dinv_ref, bh_ref, bs_ref,
                   wa_ref, wb_ref, wc_ref, o_ref):
    hh = mh_ref[...] * dinv_ref[...] + bh_ref[...]
    hs = ms_ref[...] * dinv_ref[...] + bs_ref[...]
    g = (jnp.dot(hh, wa_ref[...], preferred_element_type=jnp.float32)
         + jnp.dot(hs, wb_ref[...], preferred_element_type=jnp.float32)
         + jnp.dot(grow_ref[...], wc_ref[...],
                   preferred_element_type=jnp.float32))
    o_ref[...] = g * dinv_ref[...]


def _tc_tail0(mh, ms, grow, dinv2, bh, bs, wa, wb, wc):
    return pl.pallas_call(
        _tc_tail0_body,
        out_shape=jax.ShapeDtypeStruct((N, D), jnp.float32),
    )(mh, ms, grow, dinv2, bh, bs, wa, wb, wc)


def _tc_final_body(m_ref, dinv_ref, b_ref, o_ref):
    o_ref[...] = jnp.tanh(m_ref[...] * dinv_ref[...] + b_ref[...]) * 0.5


def _tc_final(m, dinv2, b):
    return pl.pallas_call(
        _tc_final_body,
        out_shape=jax.ShapeDtypeStruct((N, D), jnp.float32),
    )(m, dinv2, b)


def _build_csr(edge_index):
    src = edge_index[0].astype(jnp.int32)
    dst = edge_index[1].astype(jnp.int32)
    loop = jnp.arange(N, dtype=jnp.int32)
    src_f = jnp.concatenate([src, loop])
    dst_f = jnp.concatenate([dst, loop])
    ef = src_f.shape[0]
    order = jnp.argsort(dst_f)
    src_s = src_f[order]
    ds = dst_f[order]
    deg = jnp.zeros((N,), jnp.int32).at[dst_f].add(1)
    ptr = jnp.concatenate([jnp.zeros((1,), jnp.int32),
                           jnp.cumsum(deg, dtype=jnp.int32)])
    dinv = 1.0 / jnp.sqrt(deg.astype(jnp.float32))
    deg_pad = ((deg + 7) // 8) * 8
    ptr_pad = jnp.concatenate([jnp.zeros((1,), jnp.int32),
                               jnp.cumsum(deg_pad, dtype=jnp.int32)])
    marks = jnp.zeros((EP,), jnp.int32).at[ptr_pad[:N]].add(1)
    owner = jnp.cumsum(marks, dtype=jnp.int32) - 1
    pos = ptr_pad[ds] + jnp.arange(ef, dtype=jnp.int32) - ptr[ds]
    srcp = owner.at[pos].set(src_s)
    srcp = jnp.concatenate([srcp, jnp.zeros((CH,), jnp.int32)])
    ptr_full = jnp.concatenate([
        ptr_pad,
        jnp.broadcast_to(ptr_pad[N:N + 1], (PTR_LEN - (N + 1),))])
    n0_w = jnp.arange(NW, dtype=jnp.int32) * NPW
    ps = ptr_pad[jnp.minimum(loop - (loop % NPW), N)]
    cj = (ptr_pad[1:] - 1 - ps) // CH
    w_of = loop // NPW
    hist = jnp.zeros((NW * JW,), jnp.int32).at[w_of * JW + cj].add(1)
    jend = jnp.cumsum(hist.reshape(NW, JW), axis=1, dtype=jnp.int32)
    return srcp, ptr_full, jend, dinv[:, None]


def kernel(x, edge_index, head_W0, head_b0, head_W1, head_b1, head_W2, head_b2,
           head_W3, head_b3, head_W4, head_b4, skip_W0, skip_b0,
           glob_W0, glob_b0, glob_W1, glob_b1,
           tail_W0, tail_b0, tail_W1, tail_b1):
    srcp, ptr_full, jend, dinv2 = _build_csr(edge_index)
    seg = _make_segmax()

    def agg(g):
        return seg(g, srcp, ptr_full, jend)[:N]

    m = agg(_tc_first(x, head_W0, dinv2))
    for b_prev, W in [(head_b0, head_W1), (head_b1, head_W2),
                      (head_b2, head_W3), (head_b3, head_W4)]:
        m = agg(_tc_mid(m, dinv2, b_prev[None, :], W))
    m_head = m

    m_skip = agg(_tc_first(x, skip_W0, dinv2))

    mg = agg(_tc_first(x, glob_W0, dinv2))
    mg = agg(_tc_mid(mg, dinv2, glob_b0[None, :], glob_W1))
    grow = _tc_globend(mg, dinv2, glob_b1[None, :])

    wa, wb, wc = tail_W0[0:D], tail_W0[D:2 * D], tail_W0[2 * D:3 * D]
    gt = _tc_tail0(m_head, m_skip, grow, dinv2, head_b4[None, :],
                   skip_b0[None, :], wa, wb, wc)
    mt = agg(gt)
    mt = agg(_tc_mid(mt, dinv2, tail_b0[None, :], tail_W1))
    return _tc_final(mt, dinv2, tail_b1[None, :])


# speedup vs baseline: 4.6450x; 1.2074x over previous
import functools

import jax
import jax.numpy as jnp
from jax import lax
from jax.experimental import pallas as pl
from jax.experimental.pallas import tpu as pltpu
from jax.experimental.pallas import tpu_sc as plsc

N = 10000
D = 128
NB = D // 16
NW = 32
NPW = 320
NPAD = NW * NPW
CH = 256
PTR_PAD = 24
PTR_LEN = NW * NPW + PTR_PAD
EP = 160000 + N + 7 * N
MAXCH = EP // CH + 2
JW = ((MAXCH + 16 + 7) // 8) * 8
NEG = jnp.float32(-3.0e38)


@functools.cache
def _make_segmax():
    mesh = plsc.VectorSubcoreMesh(core_axis_name="c", subcore_axis_name="s")
    return functools.partial(
        pl.kernel,
        mesh=mesh,
        out_type=jax.ShapeDtypeStruct((NPAD, D), jnp.float32),
        scratch_types=[
            pltpu.VMEM((NPW + PTR_PAD,), jnp.int32),
            pltpu.VMEM((JW,), jnp.int32),
            pltpu.VMEM((CH,), jnp.int32),
            pltpu.VMEM((CH,), jnp.int32),
            pltpu.VMEM((CH, D), jnp.float32),
            pltpu.VMEM((CH, D), jnp.float32),
            pltpu.VMEM((NPW, D), jnp.float32),
            pltpu.SemaphoreType.DMA,
            pltpu.SemaphoreType.DMA,
            pltpu.SemaphoreType.DMA,
            pltpu.SemaphoreType.DMA,
        ],
    )(_segmax_body)


def _pload(ref, i):
    return ref[pl.ds(i, 16)][0]


def _segmax_body(g_hbm, srcp_hbm, ptr_hbm, jend_hbm, out_hbm,
                 ptr_v, jend_v, idx_v0, idx_v1, rows_v0, rows_v1, out_v,
                 sem_i0, sem_i1, sem_g0, sem_g1):
    wid = lax.axis_index("s") * 2 + lax.axis_index("c")
    n0 = pl.multiple_of(wid * NPW, 8)
    cnt = jnp.minimum(N - n0, NPW)
    pltpu.sync_copy(ptr_hbm.at[pl.ds(n0, NPW + PTR_PAD)], ptr_v)
    pltpu.sync_copy(jend_hbm.at[wid], jend_v)
    p_start = _pload(ptr_v, 0)
    p_end = _pload(ptr_v, cnt)
    nch = (p_end - p_start + (CH - 1)) // CH
    neg = jnp.full((16,), NEG, dtype=jnp.float32)
    sem_i = (sem_i0, sem_i1)
    sem_g = (sem_g0, sem_g1)
    idx_v = (idx_v0, idx_v1)
    rows_v = (rows_v0, rows_v1)

    def chunk_lo(c):
        return pl.multiple_of(p_start + c * CH, 8)

    def fire_idx(c, b):
        pltpu.async_copy(srcp_hbm.at[pl.ds(chunk_lo(c), CH)],
                         idx_v[b], sem_i[b])

    def wait_idx(c, b):
        pltpu.make_async_copy(srcp_hbm.at[pl.ds(chunk_lo(c), CH)],
                              idx_v[b], sem_i[b]).wait()

    def fire_gather(b):
        pltpu.async_copy(g_hbm.at[idx_v[b]], rows_v[b], sem_g[b])

    def wait_gather(b):
        pltpu.make_async_copy(g_hbm.at[idx_v[b]], rows_v[b],
                              sem_g[b]).wait()

    fire_idx(0, 0)
    wait_idx(0, 0)
    fire_gather(0)

    @pl.when(nch > 1)
    def _():
        fire_idx(1, 1)

    def step(c, b, carry):
        jc = carry[0]
        acc = carry[1:]
        lo = chunk_lo(c)
        hi = jnp.minimum(lo + CH, p_end)

        @pl.when(c < nch)
        def _():
            wait_gather(b)

        @pl.when(c + 1 < nch)
        def _():
            wait_idx(c + 1, 1 - b)
            fire_gather(1 - b)

        @pl.when(c + 2 < nch)
        def _():
            fire_idx(c + 2, b)

        def edge_max(e0, e1, a):
            def edge_body(e, aa):
                k = e - lo
                return tuple(jnp.maximum(aa[f], rows_v[b][k, pl.ds(f * 16, 16)])
                             for f in range(NB))

            return lax.fori_loop(e0, jnp.maximum(e0, e1), edge_body, a)

        je = _pload(jend_v, c)

        def node_body(j, a):
            pv = ptr_v[pl.ds(j, 16)]
            a = edge_max(jnp.maximum(pv[0], lo), pv[1], a)
            for f in range(NB):
                out_v[j, pl.ds(f * 16, 16)] = a[f]
            return tuple(neg for _ in range(NB))

        acc = lax.fori_loop(jc, je, node_body, acc)
        p0 = _pload(ptr_v, je)
        acc = edge_max(jnp.maximum(p0, lo), hi, acc)
        return (je,) + acc

    def pair_body(cp, carry):
        c0 = cp * 2
        carry = step(c0, 0, carry)
        carry = step(c0 + 1, 1, carry)
        return carry

    init = (jnp.int32(0),) + tuple(neg for _ in range(NB))
    lax.fori_loop(0, (nch + 1) // 2, pair_body, init)
    pltpu.sync_copy(out_v, out_hbm.at[pl.ds(n0, NPW)])


def _tc_first_body(x_ref, w_ref, dinv_ref, o_ref):
    o_ref[...] = jnp.dot(x_ref[...], w_ref[...],
                         preferred_element_type=jnp.float32) * dinv_ref[...]


def _tc_first(xx, W, dinv2):
    return pl.pallas_call(
        _tc_first_body,
        out_shape=jax.ShapeDtypeStruct((N, W.shape[1]), jnp.float32),
    )(xx, W, dinv2)


def _tc_mid_body(m_ref, dinv_ref, b_ref, w_ref, o_ref):
    h = m_ref[...] * dinv_ref[...] + b_ref[...]
    mu = jnp.mean(h, axis=0, keepdims=True)
    v = jnp.mean((h - mu) ** 2, axis=0, keepdims=True)
    hn = (h - mu) / jnp.sqrt(v + 1e-5)
    a = jnp.where(hn >= 0, hn, 0.02 * hn)
    o_ref[...] = jnp.dot(a, w_ref[...],
                         preferred_element_type=jnp.float32) * dinv_ref[...]


def _tc_mid(m, dinv2, b, W):
    return pl.pallas_call(
        _tc_mid_body,
        out_shape=jax.ShapeDtypeStruct((N, W.shape[1]), jnp.float32),
    )(m, dinv2, b, W)


def _tc_globend_body(m_ref, dinv_ref, b_ref, o_ref):
    h = m_ref[...] * dinv_ref[...] + b_ref[...]
    o_ref[...] = jnp.mean(h, axis=0, keepdims=True)


def _tc_globend(m, dinv2, b):
    return pl.pallas_call(
        _tc_globend_body,
        out_shape=jax.ShapeDtypeStruct((1, D), jnp.float32),
    )(m, dinv2, b)


def _tc_tail0_body(mh_ref, ms_ref, grow_ref, dinv_ref, bh_ref, bs_ref,
                   wa_ref, wb_ref, wc_ref, o_ref):
    hh = mh_ref[...] * dinv_ref[...] + bh_ref[...]
    hs = ms_ref[...] * dinv_ref[...] + bs_ref[...]
    g = (jnp.dot(hh, wa_ref[...], preferred_element_type=jnp.float32)
         + jnp.dot(hs, wb_ref[...], preferred_element_type=jnp.float32)
         + jnp.dot(grow_ref[...], wc_ref[...],
                   preferred_element_type=jnp.float32))
    o_ref[...] = g * dinv_ref[...]


def _tc_tail0(mh, ms, grow, dinv2, bh, bs, wa, wb, wc):
    return pl.pallas_call(
        _tc_tail0_body,
        out_shape=jax.ShapeDtypeStruct((N, D), jnp.float32),
    )(mh, ms, grow, dinv2, bh, bs, wa, wb, wc)


def _tc_final_body(m_ref, dinv_ref, b_ref, o_ref):
    o_ref[...] = jnp.tanh(m_ref[...] * dinv_ref[...] + b_ref[...]) * 0.5


def _tc_final(m, dinv2, b):
    return pl.pallas_call(
        _tc_final_body,
        out_shape=jax.ShapeDtypeStruct((N, D), jnp.float32),
    )(m, dinv2, b)


def _build_csr(edge_index):
    src = edge_index[0].astype(jnp.int32)
    dst = edge_index[1].astype(jnp.int32)
    loop = jnp.arange(N, dtype=jnp.int32)
    src_f = jnp.concatenate([src, loop])
    dst_f = jnp.concatenate([dst, loop])
    ef = src_f.shape[0]
    order = jnp.argsort(dst_f)
    src_s = src_f[order]
    ds = dst_f[order]
    deg = jnp.zeros((N,), jnp.int32).at[dst_f].add(1)
    ptr = jnp.concatenate([jnp.zeros((1,), jnp.int32),
                           jnp.cumsum(deg, dtype=jnp.int32)])
    dinv = 1.0 / jnp.sqrt(deg.astype(jnp.float32))
    deg_pad = ((deg + 7) // 8) * 8
    ptr_pad = jnp.concatenate([jnp.zeros((1,), jnp.int32),
                               jnp.cumsum(deg_pad, dtype=jnp.int32)])
    marks = jnp.zeros((EP,), jnp.int32).at[ptr_pad[:N]].add(1)
    owner = jnp.cumsum(marks, dtype=jnp.int32) - 1
    pos = ptr_pad[ds] + jnp.arange(ef, dtype=jnp.int32) - ptr[ds]
    srcp = owner.at[pos].set(src_s)
    srcp = jnp.concatenate([srcp, jnp.zeros((2 * CH,), jnp.int32)])
    ptr_full = jnp.concatenate([
        ptr_pad,
        jnp.broadcast_to(ptr_pad[N:N + 1], (PTR_LEN - (N + 1),))])
    n0_w = jnp.arange(NW, dtype=jnp.int32) * NPW
    ps = ptr_pad[jnp.minimum(loop - (loop % NPW), N)]
    cj = (ptr_pad[1:] - 1 - ps) // CH
    w_of = loop // NPW
    hist = jnp.zeros((NW * JW,), jnp.int32).at[w_of * JW + cj].add(1)
    jend = jnp.cumsum(hist.reshape(NW, JW), axis=1, dtype=jnp.int32)
    return srcp, ptr_full, jend, dinv[:, None]


def kernel(x, edge_index, head_W0, head_b0, head_W1, head_b1, head_W2, head_b2,
           head_W3, head_b3, head_W4, head_b4, skip_W0, skip_b0,
           glob_W0, glob_b0, glob_W1, glob_b1,
           tail_W0, tail_b0, tail_W1, tail_b1):
    srcp, ptr_full, jend, dinv2 = _build_csr(edge_index)
    seg = _make_segmax()

    def agg(g):
        return seg(g, srcp, ptr_full, jend)[:N]

    m = agg(_tc_first(x, head_W0, dinv2))
    for b_prev, W in [(head_b0, head_W1), (head_b1, head_W2),
                      (head_b2, head_W3), (head_b3, head_W4)]:
        m = agg(_tc_mid(m, dinv2, b_prev[None, :], W))
    m_head = m

    m_skip = agg(_tc_first(x, skip_W0, dinv2))

    mg = agg(_tc_first(x, glob_W0, dinv2))
    mg = agg(_tc_mid(mg, dinv2, glob_b0[None, :], glob_W1))
    grow = _tc_globend(mg, dinv2, glob_b1[None, :])

    wa, wb, wc = tail_W0[0:D], tail_W0[D:2 * D], tail_W0[2 * D:3 * D]
    gt = _tc_tail0(m_head, m_skip, grow, dinv2, head_b4[None, :],
                   skip_b0[None, :], wa, wb, wc)
    mt = agg(gt)
    mt = agg(_tc_mid(mt, dinv2, tail_b0[None, :], tail_W1))
    return _tc_final(mt, dinv2, tail_b1[None, :])


# speedup vs baseline: 18.1797x; 3.9139x over previous
import functools

import jax
import jax.numpy as jnp
from jax import lax
from jax.experimental import pallas as pl
from jax.experimental.pallas import tpu as pltpu
from jax.experimental.pallas import tpu_sc as plsc

N = 10000
D = 128
NB = D // 16
NW = 32
NPW = 320
NPAD = NW * NPW
CH = 256
PTR_PAD = 24
PTR_LEN = NW * NPW + PTR_PAD
EF = 160000 + N
MAXCH = EF // CH + 2
JW = ((MAXCH + 16 + 7) // 8) * 8
NEG = jnp.float32(-3.0e38)


@functools.cache
def _make_segmax():
    mesh = plsc.VectorSubcoreMesh(core_axis_name="c", subcore_axis_name="s")
    return functools.partial(
        pl.kernel,
        mesh=mesh,
        out_type=jax.ShapeDtypeStruct((NPAD, D), jnp.float32),
        scratch_types=[
            pltpu.VMEM((NPW + PTR_PAD,), jnp.int32),
            pltpu.VMEM((JW,), jnp.int32),
            pltpu.VMEM((CH,), jnp.int32),
            pltpu.VMEM((CH,), jnp.int32),
            pltpu.VMEM((CH, D), jnp.float32),
            pltpu.VMEM((CH, D), jnp.float32),
            pltpu.VMEM((NPW, D), jnp.float32),
            pltpu.SemaphoreType.DMA,
            pltpu.SemaphoreType.DMA,
            pltpu.SemaphoreType.DMA,
            pltpu.SemaphoreType.DMA,
        ],
    )(_segmax_body)


def _pload(ref, i):
    return ref[pl.ds(i, 16)][0]


def _segmax_body(g_hbm, srcp_hbm, ptr_hbm, jend_hbm, out_hbm,
                 ptr_v, jend_v, idx_v0, idx_v1, rows_v0, rows_v1, out_v,
                 sem_i0, sem_i1, sem_g0, sem_g1):
    wid = lax.axis_index("s") * 2 + lax.axis_index("c")
    n0 = pl.multiple_of(wid * NPW, 8)
    cnt = jnp.minimum(N - n0, NPW)
    pltpu.sync_copy(ptr_hbm.at[pl.ds(n0, NPW + PTR_PAD)], ptr_v)
    pltpu.sync_copy(jend_hbm.at[wid], jend_v)
    p_start = (_pload(ptr_v, 0) // 8) * 8
    p_end = _pload(ptr_v, cnt)
    nch = (p_end - p_start + (CH - 1)) // CH
    neg = jnp.full((16,), NEG, dtype=jnp.float32)
    sem_i = (sem_i0, sem_i1)
    sem_g = (sem_g0, sem_g1)
    idx_v = (idx_v0, idx_v1)
    rows_v = (rows_v0, rows_v1)

    def chunk_lo(c):
        return pl.multiple_of(p_start + c * CH, 8)

    def fire_idx(c, b):
        pltpu.async_copy(srcp_hbm.at[pl.ds(chunk_lo(c), CH)],
                         idx_v[b], sem_i[b])

    def wait_idx(c, b):
        pltpu.make_async_copy(srcp_hbm.at[pl.ds(chunk_lo(c), CH)],
                              idx_v[b], sem_i[b]).wait()

    def fire_gather(b):
        pltpu.async_copy(g_hbm.at[idx_v[b]], rows_v[b], sem_g[b])

    def wait_gather(b):
        pltpu.make_async_copy(g_hbm.at[idx_v[b]], rows_v[b],
                              sem_g[b]).wait()

    fire_idx(0, 0)
    wait_idx(0, 0)
    fire_gather(0)

    @pl.when(nch > 1)
    def _():
        fire_idx(1, 1)

    def step(c, b, carry):
        jc = carry[0]
        acc = carry[1:]
        lo = chunk_lo(c)
        hi = jnp.minimum(lo + CH, p_end)

        @pl.when(c < nch)
        def _():
            wait_gather(b)

        @pl.when(c + 1 < nch)
        def _():
            wait_idx(c + 1, 1 - b)
            fire_gather(1 - b)

        @pl.when(c + 2 < nch)
        def _():
            fire_idx(c + 2, b)

        def edge_max(e0, e1, a):
            def edge_body(e, aa):
                k = e - lo
                return tuple(jnp.maximum(aa[f], rows_v[b][k, pl.ds(f * 16, 16)])
                             for f in range(NB))

            return lax.fori_loop(e0, jnp.maximum(e0, e1), edge_body, a)

        je = _pload(jend_v, c)

        def node_body(j, a):
            pv = ptr_v[pl.ds(j, 16)]
            a = edge_max(jnp.maximum(pv[0], lo), pv[1], a)
            for f in range(NB):
                out_v[j, pl.ds(f * 16, 16)] = a[f]
            return tuple(neg for _ in range(NB))

        acc = lax.fori_loop(jc, je, node_body, acc)
        p0 = _pload(ptr_v, je)
        acc = edge_max(jnp.maximum(p0, lo), hi, acc)
        return (je,) + acc

    def pair_body(cp, carry):
        c0 = cp * 2
        carry = step(c0, 0, carry)
        carry = step(c0 + 1, 1, carry)
        return carry

    init = (jnp.int32(0),) + tuple(neg for _ in range(NB))
    lax.fori_loop(0, (nch + 1) // 2, pair_body, init)
    pltpu.sync_copy(out_v, out_hbm.at[pl.ds(n0, NPW)])


def _tc_first_body(x_ref, w_ref, dinv_ref, o_ref):
    o_ref[...] = jnp.dot(x_ref[...], w_ref[...],
                         preferred_element_type=jnp.float32) * dinv_ref[...]


def _tc_first(xx, W, dinv2):
    return pl.pallas_call(
        _tc_first_body,
        out_shape=jax.ShapeDtypeStruct((N, W.shape[1]), jnp.float32),
    )(xx, W, dinv2)


def _tc_mid_body(m_ref, dinv_ref, b_ref, w_ref, o_ref):
    h = m_ref[...] * dinv_ref[...] + b_ref[...]
    mu = jnp.mean(h, axis=0, keepdims=True)
    v = jnp.mean((h - mu) ** 2, axis=0, keepdims=True)
    hn = (h - mu) / jnp.sqrt(v + 1e-5)
    a = jnp.where(hn >= 0, hn, 0.02 * hn)
    o_ref[...] = jnp.dot(a, w_ref[...],
                         preferred_element_type=jnp.float32) * dinv_ref[...]


def _tc_mid(m, dinv2, b, W):
    return pl.pallas_call(
        _tc_mid_body,
        out_shape=jax.ShapeDtypeStruct((N, W.shape[1]), jnp.float32),
    )(m, dinv2, b, W)


def _tc_globend_body(m_ref, dinv_ref, b_ref, o_ref):
    h = m_ref[...] * dinv_ref[...] + b_ref[...]
    o_ref[...] = jnp.mean(h, axis=0, keepdims=True)


def _tc_globend(m, dinv2, b):
    return pl.pallas_call(
        _tc_globend_body,
        out_shape=jax.ShapeDtypeStruct((1, D), jnp.float32),
    )(m, dinv2, b)


def _tc_tail0_body(mh_ref, ms_ref, grow_ref, dinv_ref, bh_ref, bs_ref,
                   wa_ref, wb_ref, wc_ref, o_ref):
    hh = mh_ref[...] * dinv_ref[...] + bh_ref[...]
    hs = ms_ref[...] * dinv_ref[...] + bs_ref[...]
    g = (jnp.dot(hh, wa_ref[...], preferred_element_type=jnp.float32)
         + jnp.dot(hs, wb_ref[...], preferred_element_type=jnp.float32)
         + jnp.dot(grow_ref[...], wc_ref[...],
                   preferred_element_type=jnp.float32))
    o_ref[...] = g * dinv_ref[...]


def _tc_tail0(mh, ms, grow, dinv2, bh, bs, wa, wb, wc):
    return pl.pallas_call(
        _tc_tail0_body,
        out_shape=jax.ShapeDtypeStruct((N, D), jnp.float32),
    )(mh, ms, grow, dinv2, bh, bs, wa, wb, wc)


def _tc_final_body(m_ref, dinv_ref, b_ref, o_ref):
    o_ref[...] = jnp.tanh(m_ref[...] * dinv_ref[...] + b_ref[...]) * 0.5


def _tc_final(m, dinv2, b):
    return pl.pallas_call(
        _tc_final_body,
        out_shape=jax.ShapeDtypeStruct((N, D), jnp.float32),
    )(m, dinv2, b)


def _build_csr(edge_index):
    src = edge_index[0].astype(jnp.int32)
    dst = edge_index[1].astype(jnp.int32)
    loop = jnp.arange(N, dtype=jnp.int32)
    src_f = jnp.concatenate([src, loop])
    dst_f = jnp.concatenate([dst, loop])
    ef = src_f.shape[0]
    order = jnp.argsort(dst_f)
    src_s = src_f[order]
    ds = dst_f[order]
    deg = jnp.zeros((N,), jnp.int32).at[dst_f].add(1)
    ptr = jnp.concatenate([jnp.zeros((1,), jnp.int32),
                           jnp.cumsum(deg, dtype=jnp.int32)])
    dinv = 1.0 / jnp.sqrt(deg.astype(jnp.float32))
    srcp = jnp.concatenate([src_s, jnp.zeros((2 * CH,), jnp.int32)])
    ptr_full = jnp.concatenate([
        ptr,
        jnp.broadcast_to(ptr[N:N + 1], (PTR_LEN - (N + 1),))])
    lo0 = (ptr[jnp.minimum(loop - (loop % NPW), N)] // 8) * 8
    cj = (ptr[1:] - 1 - lo0) // CH
    w_of = loop // NPW
    hist = jnp.zeros((NW * JW,), jnp.int32).at[w_of * JW + cj].add(1)
    jend = jnp.cumsum(hist.reshape(NW, JW), axis=1, dtype=jnp.int32)
    return srcp, ptr_full, jend, dinv[:, None]


def kernel(x, edge_index, head_W0, head_b0, head_W1, head_b1, head_W2, head_b2,
           head_W3, head_b3, head_W4, head_b4, skip_W0, skip_b0,
           glob_W0, glob_b0, glob_W1, glob_b1,
           tail_W0, tail_b0, tail_W1, tail_b1):
    srcp, ptr_full, jend, dinv2 = _build_csr(edge_index)
    seg = _make_segmax()

    def agg(g):
        return seg(g, srcp, ptr_full, jend)[:N]

    m = agg(_tc_first(x, head_W0, dinv2))
    for b_prev, W in [(head_b0, head_W1), (head_b1, head_W2),
                      (head_b2, head_W3), (head_b3, head_W4)]:
        m = agg(_tc_mid(m, dinv2, b_prev[None, :], W))
    m_head = m

    m_skip = agg(_tc_first(x, skip_W0, dinv2))

    mg = agg(_tc_first(x, glob_W0, dinv2))
    mg = agg(_tc_mid(mg, dinv2, glob_b0[None, :], glob_W1))
    grow = _tc_globend(mg, dinv2, glob_b1[None, :])

    wa, wb, wc = tail_W0[0:D], tail_W0[D:2 * D], tail_W0[2 * D:3 * D]
    gt = _tc_tail0(m_head, m_skip, grow, dinv2, head_b4[None, :],
                   skip_b0[None, :], wa, wb, wc)
    mt = agg(gt)
    mt = agg(_tc_mid(mt, dinv2, tail_b0[None, :], tail_W1))
    return _tc_final(mt, dinv2, tail_b1[None, :])


# speedup vs baseline: 18.4540x; 1.0151x over previous
import functools

import jax
import jax.numpy as jnp
from jax import lax
from jax.experimental import pallas as pl
from jax.experimental.pallas import tpu as pltpu
from jax.experimental.pallas import tpu_sc as plsc

N = 10000
D = 128
NB = D // 16
NW = 32
NPW = 320
NPAD = NW * NPW
CH = 256
PTR_PAD = 24
PTR_LEN = NW * NPW + PTR_PAD
EF = 160000 + N
MAXCH = EF // CH + 2
JW = ((MAXCH + 16 + 7) // 8) * 8
NEG = jnp.float32(-3.0e38)


@functools.cache
def _make_segmax():
    mesh = plsc.VectorSubcoreMesh(core_axis_name="c", subcore_axis_name="s")
    return functools.partial(
        pl.kernel,
        mesh=mesh,
        out_type=jax.ShapeDtypeStruct((NPAD, D), jnp.float32),
        scratch_types=[
            pltpu.VMEM((NPW + PTR_PAD,), jnp.int32),
            pltpu.VMEM((JW,), jnp.int32),
            pltpu.VMEM((CH,), jnp.int32),
            pltpu.VMEM((CH,), jnp.int32),
            pltpu.VMEM((CH, D), jnp.float32),
            pltpu.VMEM((CH, D), jnp.float32),
            pltpu.VMEM((NPW, D), jnp.float32),
            pltpu.SemaphoreType.DMA,
            pltpu.SemaphoreType.DMA,
            pltpu.SemaphoreType.DMA,
            pltpu.SemaphoreType.DMA,
        ],
    )(_segmax_body)


def _pload(ref, i):
    return ref[pl.ds(i, 16)][0]


def _segmax_body(g_hbm, srcp_hbm, ptr_hbm, jend_hbm, out_hbm,
                 ptr_v, jend_v, idx_v0, idx_v1, rows_v0, rows_v1, out_v,
                 sem_i0, sem_i1, sem_g0, sem_g1):
    wid = lax.axis_index("s") * 2 + lax.axis_index("c")
    n0 = pl.multiple_of(wid * NPW, 8)
    cnt = jnp.minimum(N - n0, NPW)
    pltpu.sync_copy(ptr_hbm.at[pl.ds(n0, NPW + PTR_PAD)], ptr_v)
    pltpu.sync_copy(jend_hbm.at[wid], jend_v)
    p_start = (_pload(ptr_v, 0) // 8) * 8
    p_end = _pload(ptr_v, cnt)
    nch = (p_end - p_start + (CH - 1)) // CH
    neg = jnp.full((16,), NEG, dtype=jnp.float32)
    sem_i = (sem_i0, sem_i1)
    sem_g = (sem_g0, sem_g1)
    idx_v = (idx_v0, idx_v1)
    rows_v = (rows_v0, rows_v1)

    def chunk_lo(c):
        return pl.multiple_of(p_start + c * CH, 8)

    def fire_idx(c, b):
        pltpu.async_copy(srcp_hbm.at[pl.ds(chunk_lo(c), CH)],
                         idx_v[b], sem_i[b])

    def wait_idx(c, b):
        pltpu.make_async_copy(srcp_hbm.at[pl.ds(chunk_lo(c), CH)],
                              idx_v[b], sem_i[b]).wait()

    def fire_gather(b):
        pltpu.async_copy(g_hbm.at[idx_v[b]], rows_v[b], sem_g[b])

    def wait_gather(b):
        pltpu.make_async_copy(g_hbm.at[idx_v[b]], rows_v[b],
                              sem_g[b]).wait()

    fire_idx(0, 0)
    wait_idx(0, 0)
    fire_gather(0)

    @pl.when(nch > 1)
    def _():
        fire_idx(1, 1)

    def step(c, b, carry):
        jc = carry[0]
        acc = carry[1:]
        lo = chunk_lo(c)
        hi = jnp.minimum(lo + CH, p_end)

        @pl.when(c < nch)
        def _():
            wait_gather(b)

        @pl.when(c + 1 < nch)
        def _():
            wait_idx(c + 1, 1 - b)
            fire_gather(1 - b)

        @pl.when(c + 2 < nch)
        def _():
            fire_idx(c + 2, b)

        def edge_max(e0, e1, a):
            def edge_body(e, aa):
                k = e - lo
                return tuple(jnp.maximum(aa[f], rows_v[b][k, pl.ds(f * 16, 16)])
                             for f in range(NB))

            return lax.fori_loop(e0, jnp.maximum(e0, e1), edge_body, a)

        je = _pload(jend_v, c)

        def node_body(j, a):
            pv = ptr_v[pl.ds(j, 16)]
            a = edge_max(jnp.maximum(pv[0], lo), pv[1], a)
            for f in range(NB):
                out_v[j, pl.ds(f * 16, 16)] = a[f]
            return tuple(neg for _ in range(NB))

        acc = lax.fori_loop(jc, je, node_body, acc)
        p0 = _pload(ptr_v, je)
        acc = edge_max(jnp.maximum(p0, lo), hi, acc)
        return (je,) + acc

    def pair_body(cp, carry):
        c0 = cp * 2
        carry = step(c0, 0, carry)
        carry = step(c0 + 1, 1, carry)
        return carry

    init = (jnp.int32(0),) + tuple(neg for _ in range(NB))
    lax.fori_loop(0, (nch + 1) // 2, pair_body, init)
    pltpu.sync_copy(out_v, out_hbm.at[pl.ds(n0, NPW)])


def _tc_first_body(x_ref, w_ref, dinv_ref, o_ref):
    o_ref[...] = jnp.dot(x_ref[...], w_ref[...],
                         preferred_element_type=jnp.float32) * dinv_ref[...]


def _tc_first(xx, W, dinv2):
    return pl.pallas_call(
        _tc_first_body,
        out_shape=jax.ShapeDtypeStruct((N, W.shape[1]), jnp.float32),
    )(xx, W, dinv2)


def _tc_mid_body(m_ref, dinv_ref, b_ref, w_ref, o_ref):
    h = m_ref[...] * dinv_ref[...] + b_ref[...]
    mu = jnp.mean(h, axis=0, keepdims=True)
    v = jnp.mean((h - mu) ** 2, axis=0, keepdims=True)
    hn = (h - mu) / jnp.sqrt(v + 1e-5)
    a = jnp.where(hn >= 0, hn, 0.02 * hn)
    o_ref[...] = jnp.dot(a, w_ref[...],
                         preferred_element_type=jnp.float32) * dinv_ref[...]


def _tc_mid(m, dinv2, b, W):
    return pl.pallas_call(
        _tc_mid_body,
        out_shape=jax.ShapeDtypeStruct((N, W.shape[1]), jnp.float32),
    )(m, dinv2, b, W)


def _tc_globend_body(m_ref, dinv_ref, b_ref, o_ref):
    h = m_ref[...] * dinv_ref[...] + b_ref[...]
    o_ref[...] = jnp.mean(h, axis=0, keepdims=True)


def _tc_globend(m, dinv2, b):
    return pl.pallas_call(
        _tc_globend_body,
        out_shape=jax.ShapeDtypeStruct((1, D), jnp.float32),
    )(m, dinv2, b)


def _tc_tail0_body(mh_ref, ms_ref, grow_ref, dinv_ref, bh_ref, bs_ref,
                   wa_ref, wb_ref, wc_ref, o_ref):
    hh = mh_ref[...] * dinv_ref[...] + bh_ref[...]
    hs = ms_ref[...] * dinv_ref[...] + bs_ref[...]
    g = (jnp.dot(hh, wa_ref[...], preferred_element_type=jnp.float32)
         + jnp.dot(hs, wb_ref[...], preferred_element_type=jnp.float32)
         + jnp.dot(grow_ref[...], wc_ref[...],
                   preferred_element_type=jnp.float32))
    o_ref[...] = g * dinv_ref[...]


def _tc_tail0(mh, ms, grow, dinv2, bh, bs, wa, wb, wc):
    return pl.pallas_call(
        _tc_tail0_body,
        out_shape=jax.ShapeDtypeStruct((N, D), jnp.float32),
    )(mh, ms, grow, dinv2, bh, bs, wa, wb, wc)


def _tc_final_body(m_ref, dinv_ref, b_ref, o_ref):
    o_ref[...] = jnp.tanh(m_ref[...] * dinv_ref[...] + b_ref[...]) * 0.5


def _tc_final(m, dinv2, b):
    return pl.pallas_call(
        _tc_final_body,
        out_shape=jax.ShapeDtypeStruct((N, D), jnp.float32),
    )(m, dinv2, b)


def _build_csr(edge_index):
    src = edge_index[0].astype(jnp.int32)
    dst = edge_index[1].astype(jnp.int32)
    loop = jnp.arange(N, dtype=jnp.int32)
    src_f = jnp.concatenate([src, loop])
    dst_f = jnp.concatenate([dst, loop])
    _, src_s = lax.sort_key_val(dst_f, src_f)
    deg = jnp.zeros((N,), jnp.int32).at[dst_f].add(1)
    ptr = jnp.concatenate([jnp.zeros((1,), jnp.int32),
                           jnp.cumsum(deg, dtype=jnp.int32)])
    dinv = 1.0 / jnp.sqrt(deg.astype(jnp.float32))
    srcp = jnp.concatenate([src_s, jnp.zeros((2 * CH,), jnp.int32)])
    ptr_full = jnp.concatenate([
        ptr,
        jnp.broadcast_to(ptr[N:N + 1], (PTR_LEN - (N + 1),))])
    lo0 = (ptr[jnp.minimum(loop - (loop % NPW), N)] // 8) * 8
    cj = (ptr[1:] - 1 - lo0) // CH
    w_of = loop // NPW
    hist = jnp.zeros((NW * JW,), jnp.int32).at[w_of * JW + cj].add(1)
    jend = jnp.cumsum(hist.reshape(NW, JW), axis=1, dtype=jnp.int32)
    return srcp, ptr_full, jend, dinv[:, None]


def kernel(x, edge_index, head_W0, head_b0, head_W1, head_b1, head_W2, head_b2,
           head_W3, head_b3, head_W4, head_b4, skip_W0, skip_b0,
           glob_W0, glob_b0, glob_W1, glob_b1,
           tail_W0, tail_b0, tail_W1, tail_b1):
    srcp, ptr_full, jend, dinv2 = _build_csr(edge_index)
    seg = _make_segmax()

    def agg(g):
        return seg(g, srcp, ptr_full, jend)[:N]

    m = agg(_tc_first(x, head_W0, dinv2))
    for b_prev, W in [(head_b0, head_W1), (head_b1, head_W2),
                      (head_b2, head_W3), (head_b3, head_W4)]:
        m = agg(_tc_mid(m, dinv2, b_prev[None, :], W))
    m_head = m

    m_skip = agg(_tc_first(x, skip_W0, dinv2))

    mg = agg(_tc_first(x, glob_W0, dinv2))
    mg = agg(_tc_mid(mg, dinv2, glob_b0[None, :], glob_W1))
    grow = _tc_globend(mg, dinv2, glob_b1[None, :])

    wa, wb, wc = tail_W0[0:D], tail_W0[D:2 * D], tail_W0[2 * D:3 * D]
    gt = _tc_tail0(m_head, m_skip, grow, dinv2, head_b4[None, :],
                   skip_b0[None, :], wa, wb, wc)
    mt = agg(gt)
    mt = agg(_tc_mid(mt, dinv2, tail_b0[None, :], tail_W1))
    return _tc_final(mt, dinv2, tail_b1[None, :])
